# baseline (speedup 1.0000x reference)
import functools

import jax
import jax.numpy as jnp
import numpy as np
from jax import lax
from jax.experimental import pallas as pl
from jax.experimental.pallas import tpu as pltpu
from jax.experimental.pallas import tpu_sc as plsc

SMALLEST_FLOAT = float(np.finfo(np.float32).tiny) / 100.0

_SC_C = 2
_SC_D = 4


@jax.jit
def kernel(logits, labels, candidate_ids):
    B, N = logits.shape
    info = plsc.get_sparse_core_info()
    NC, NS, L = info.num_cores, info.num_subcores, info.num_lanes
    NW = NC * NS
    C = _SC_C
    D = _SC_D
    rows_per_w = B // NW
    n_groups = rows_per_w // C
    n_q = N // (4 * L)
    n_t = N // L

    mesh = plsc.VectorSubcoreMesh(core_axis_name="c", subcore_axis_name="s")

    @functools.partial(
        pl.kernel,
        mesh=mesh,
        out_type=jax.ShapeDtypeStruct((B, N), jnp.float32),
        scratch_types=[
            pltpu.VMEM((D, C, N), jnp.float32),
            pltpu.VMEM((D, C, N), jnp.float32),
            pltpu.VMEM((D, C, N), jnp.float32),
            pltpu.VMEM((N,), jnp.int32),
        ] + [pltpu.SemaphoreType.DMA] * (2 * D),
        compiler_params=pltpu.CompilerParams(
            needs_layout_passes=False, use_tc_tiling_on_sc=False),
    )
    def sc_body(logits_hbm, labels_hbm, cids_hbm, out_hbm,
                lab_v, log_v, out_v, cids_v, *sems):
        wid = lax.axis_index("s") * NC + lax.axis_index("c")
        base = wid * rows_per_w
        sin = list(sems[:D])
        sout = list(sems[D:])
        iota = lax.broadcasted_iota(jnp.int32, (L,), 0)
        neg_inf = jnp.full((L,), -jnp.inf, jnp.float32)
        zero_i = jnp.zeros((L,), jnp.int32)
        big_i = jnp.full((L,), N, jnp.int32)
        sf = jnp.full((L,), SMALLEST_FLOAT, jnp.float32)
        zf = jnp.zeros((L,), jnp.float32)

        def start_in(g, p):
            r0 = base + g * C
            pltpu.async_copy(labels_hbm.at[pl.ds(r0, C)], lab_v.at[p], sin[p])
            pltpu.async_copy(logits_hbm.at[pl.ds(r0, C)], log_v.at[p], sin[p])

        def wait_in(g, p):
            r0 = base + g * C
            pltpu.make_async_copy(
                labels_hbm.at[pl.ds(r0, C)], lab_v.at[p], sin[p]).wait()
            pltpu.make_async_copy(
                logits_hbm.at[pl.ds(r0, C)], log_v.at[p], sin[p]).wait()

        def wait_out(g, p):
            r0 = base + g * C
            pltpu.make_async_copy(
                out_v.at[p], out_hbm.at[pl.ds(r0, C)], sout[p]).wait()

        for p0 in range(D):
            start_in(p0, p0)
        pltpu.sync_copy(cids_hbm, cids_v)

        def tick(g, p):
            r0 = base + g * C
            wait_in(g, p)
            pl.when(g >= D)(lambda: wait_out(g - D, p))
            pos = []
            for c in range(C):
                def p1(t, carry):
                    m0, m1, m2, m3, i0, i1, i2, i3 = carry
                    ms = [m0, m1, m2, m3]
                    js = [i0, i1, i2, i3]
                    tL = t * (4 * L)
                    for j in range(4):
                        off = tL + j * L
                        v = lab_v[p, c, pl.ds(off, L)]
                        cond = v > ms[j]
                        js[j] = jnp.where(cond, iota + off, js[j])
                        ms[j] = jnp.maximum(ms[j], v)
                    return tuple(ms) + tuple(js)

                m0, m1, m2, m3, i0, i1, i2, i3 = plsc.parallel_loop(
                    0, n_q, unroll=4,
                    carry=(neg_inf, neg_inf, neg_inf, neg_inf,
                           zero_i, zero_i, zero_i, zero_i))(p1)
                mm = jnp.maximum(jnp.maximum(m0, m1), jnp.maximum(m2, m3))
                rowmax = plsc.cummax(lax.rev(plsc.cummax(mm), (0,)))
                cand = jnp.minimum(
                    jnp.minimum(jnp.where(m0 == rowmax, i0, big_i),
                                jnp.where(m1 == rowmax, i1, big_i)),
                    jnp.minimum(jnp.where(m2 == rowmax, i2, big_i),
                                jnp.where(m3 == rowmax, i3, big_i)))
                idx_vec = -plsc.cummax(lax.rev(plsc.cummax(-cand), (0,)))
                pos.append(plsc.load_gather(cids_v, [idx_vec]))

            def p2(t):
                off = t * L
                cid = cids_v[pl.ds(off, L)]
                for c in range(C):
                    val = jnp.where(cid == pos[c], sf, zf)
                    out_v[p, c, pl.ds(off, L)] = log_v[p, c, pl.ds(off, L)] + val

            plsc.parallel_loop(0, n_t, unroll=4)(p2)
            pltpu.async_copy(out_v.at[p], out_hbm.at[pl.ds(r0, C)], sout[p])
            pl.when(g + D < n_groups)(lambda: start_in(g + D, p))

        def turn(h, _):
            for p0 in range(D):
                tick(D * h + p0, p0)
            return 0

        lax.fori_loop(0, n_groups // D, turn, 0)
        for p0 in range(D):
            wait_out(n_groups - D + p0, p0)

    return sc_body(logits, labels, candidate_ids)


# speedup vs baseline: 2.8389x; 2.8389x over previous
import functools

import jax
import jax.numpy as jnp
import numpy as np
from jax import lax
from jax.experimental import pallas as pl
from jax.experimental.pallas import tpu as pltpu
from jax.experimental.pallas import tpu_sc as plsc

SMALLEST_FLOAT = float(np.finfo(np.float32).tiny) / 100.0

_SC_C = 2
_SC_D = 4


@jax.jit
def kernel(logits, labels, candidate_ids):
    B, N = logits.shape
    info = plsc.get_sparse_core_info()
    NC, NS, L = info.num_cores, info.num_subcores, info.num_lanes
    NW = NC * NS
    C = _SC_C
    D = _SC_D
    rows_per_w = B // NW
    n_groups = rows_per_w // C
    n_q = N // (4 * L)
    n_t = N // L

    mesh = plsc.VectorSubcoreMesh(core_axis_name="c", subcore_axis_name="s")

    @functools.partial(
        pl.kernel,
        mesh=mesh,
        out_type=jax.ShapeDtypeStruct((B, N), jnp.float32),
        scratch_types=[
            pltpu.VMEM((D, C, N), jnp.float32),
            pltpu.VMEM((D, C, N), jnp.float32),
            pltpu.VMEM((D, C, N), jnp.float32),
            pltpu.VMEM((N,), jnp.int32),
        ] + [pltpu.SemaphoreType.DMA] * (2 * D),
        compiler_params=pltpu.CompilerParams(needs_layout_passes=False),
    )
    def sc_body(logits_hbm, labels_hbm, cids_hbm, out_hbm,
                lab_v, log_v, out_v, cids_v, *sems):
        wid = lax.axis_index("s") * NC + lax.axis_index("c")
        base = wid * rows_per_w
        sin = list(sems[:D])
        sout = list(sems[D:])
        iota = lax.broadcasted_iota(jnp.int32, (L,), 0)
        neg_inf = jnp.full((L,), -jnp.inf, jnp.float32)
        zero_i = jnp.zeros((L,), jnp.int32)
        big_i = jnp.full((L,), N, jnp.int32)
        sf = jnp.full((L,), SMALLEST_FLOAT, jnp.float32)
        zf = jnp.zeros((L,), jnp.float32)

        def start_in(g, p):
            r0 = base + g * C
            pltpu.async_copy(labels_hbm.at[pl.ds(r0, C)], lab_v.at[p], sin[p])
            pltpu.async_copy(logits_hbm.at[pl.ds(r0, C)], log_v.at[p], sin[p])

        def wait_in(g, p):
            r0 = base + g * C
            pltpu.make_async_copy(
                labels_hbm.at[pl.ds(r0, C)], lab_v.at[p], sin[p]).wait()
            pltpu.make_async_copy(
                logits_hbm.at[pl.ds(r0, C)], log_v.at[p], sin[p]).wait()

        def wait_out(g, p):
            r0 = base + g * C
            pltpu.make_async_copy(
                out_v.at[p], out_hbm.at[pl.ds(r0, C)], sout[p]).wait()

        for p0 in range(D):
            start_in(p0, p0)
        pltpu.sync_copy(cids_hbm, cids_v)

        def tick(g, p):
            r0 = base + g * C
            wait_in(g, p)
            pl.when(g >= D)(lambda: wait_out(g - D, p))
            pos = []
            for c in range(C):
                def p1(t, carry):
                    m0, m1, m2, m3, i0, i1, i2, i3 = carry
                    ms = [m0, m1, m2, m3]
                    js = [i0, i1, i2, i3]
                    tL = t * (4 * L)
                    for j in range(4):
                        off = tL + j * L
                        v = lab_v[p, c, pl.ds(off, L)]
                        cond = v > ms[j]
                        js[j] = jnp.where(cond, iota + off, js[j])
                        ms[j] = jnp.maximum(ms[j], v)
                    return tuple(ms) + tuple(js)

                m0, m1, m2, m3, i0, i1, i2, i3 = plsc.parallel_loop(
                    0, n_q, unroll=4,
                    carry=(neg_inf, neg_inf, neg_inf, neg_inf,
                           zero_i, zero_i, zero_i, zero_i))(p1)
                mm = jnp.maximum(jnp.maximum(m0, m1), jnp.maximum(m2, m3))
                rowmax = plsc.cummax(lax.rev(plsc.cummax(mm), (0,)))
                cand = jnp.minimum(
                    jnp.minimum(jnp.where(m0 == rowmax, i0, big_i),
                                jnp.where(m1 == rowmax, i1, big_i)),
                    jnp.minimum(jnp.where(m2 == rowmax, i2, big_i),
                                jnp.where(m3 == rowmax, i3, big_i)))
                idx_vec = -plsc.cummax(lax.rev(plsc.cummax(-cand), (0,)))
                pos.append(plsc.load_gather(cids_v, [idx_vec]))

            def p2(t):
                off = t * L
                cid = cids_v[pl.ds(off, L)]
                for c in range(C):
                    val = jnp.where(cid == pos[c], sf, zf)
                    out_v[p, c, pl.ds(off, L)] = log_v[p, c, pl.ds(off, L)] + val

            plsc.parallel_loop(0, n_t, unroll=4)(p2)
            pltpu.async_copy(out_v.at[p], out_hbm.at[pl.ds(r0, C)], sout[p])
            pl.when(g + D < n_groups)(lambda: start_in(g + D, p))

        def turn(h, _):
            for p0 in range(D):
                tick(D * h + p0, p0)
            return 0

        lax.fori_loop(0, n_groups // D, turn, 0)
        for p0 in range(D):
            wait_out(n_groups - D + p0, p0)

    return sc_body(logits, labels, candidate_ids)
